# baseline (speedup 1.0000x reference)
import functools

import jax
import jax.numpy as jnp
from jax import lax
from jax.experimental import pallas as pl
from jax.experimental.pallas import tpu as pltpu
from jax.experimental.pallas import tpu_sc as plsc

NC = 2
NS = 16
NW = NC * NS
L = 16
CHUNK = 16
NH = 7
RPC = CHUNK * NH


def _unpack_bf16_pair(w):
    lo = lax.bitcast_convert_type(
        lax.shift_left(w, jnp.int32(16)), jnp.float32)
    hi = lax.bitcast_convert_type(w, jnp.float32)
    return lo, hi


@functools.lru_cache(maxsize=None)
def _make_partial_kernel(nbatch: int, npw0: int, npw1: int, d: int,
                         vlast: int):
    dw = d
    st0 = npw0 // CHUNK
    st1 = npw1 // CHUNK
    aw0 = npw0 * NH
    aw1 = npw1 * NH
    awords = max(aw0, aw1)
    assert st0 % 2 == 0 and st1 % 2 == 0
    assert aw0 % 8 == 0 and aw1 % 8 == 0

    mesh = plsc.VectorSubcoreMesh(core_axis_name="c", subcore_axis_name="s")

    @functools.partial(
        pl.kernel,
        mesh=mesh,
        out_type=jax.ShapeDtypeStruct((NW, L), jnp.float32),
        scratch_types=[
            pltpu.VMEM((awords,), jnp.int32),
            pltpu.VMEM((RPC, dw), jnp.int32),
            pltpu.VMEM((RPC, dw), jnp.int32),
            pltpu.VMEM((L,), jnp.float32),
            pltpu.SemaphoreType.DMA,
            pltpu.SemaphoreType.DMA,
        ],
    )
    def nh_partial(table, adjw, out, adjv, rows0, rows1, accv, sem0, sem1):
        c = lax.axis_index("c")
        s = lax.axis_index("s")
        wid = c * NS + s
        nsteps = jnp.where(c == 0, st0, st1)
        giters = jnp.where(c == 0, st0 // 2, st1 // 2)
        off = pl.multiple_of(
            jnp.where(c == 0, s * aw0, NS * aw0 + s * aw1), 8)

        @pl.when(wid < NW - 1)
        def _():
            pltpu.sync_copy(adjw.at[pl.ds(off, awords)], adjv)

        @pl.when(wid == NW - 1)
        def _():
            pltpu.sync_copy(
                adjw.at[pl.ds(off, vlast)], adjv.at[pl.ds(0, vlast)])
            zero = jnp.zeros((L,), jnp.int32)
            for t in range(vlast, awords, L):
                adjv[pl.ds(t, L)] = zero

        def copy(s, buf, sem):
            off = pl.multiple_of(s * RPC, 8)
            return pltpu.make_async_copy(
                table.at[adjv.at[pl.ds(off, RPC)]], buf, sem)

        copy(0, rows0, sem0).start()
        copy(1, rows1, sem1).start()

        def chunk(buf, accs):
            def node(i, accs):
                base = i * NH

                def jstep(j, accs):
                    a0, a1 = accs
                    col = pl.multiple_of(j * L, L)
                    c_lo, c_hi = _unpack_bf16_pair(buf[base, pl.ds(col, L)])
                    ds_lo = []
                    ds_hi = []
                    for k in range(1, NH):
                        n_lo, n_hi = _unpack_bf16_pair(
                            buf[base + k, pl.ds(col, L)])
                        ds_lo.append(jnp.abs(c_lo - n_lo))
                        ds_hi.append(jnp.abs(c_hi - n_hi))
                    t_lo = ((ds_lo[0] + ds_lo[1]) + (ds_lo[2] + ds_lo[3])
                            + (ds_lo[4] + ds_lo[5]))
                    t_hi = ((ds_hi[0] + ds_hi[1]) + (ds_hi[2] + ds_hi[3])
                            + (ds_hi[4] + ds_hi[5]))
                    return (a0 + t_lo, a1 + t_hi)

                return lax.fori_loop(0, dw // L, jstep, accs)
            return lax.fori_loop(0, CHUNK, node, accs)

        def gstep(g, accs):
            s0 = 2 * g
            copy(s0, rows0, sem0).wait()
            accs = chunk(rows0, accs)

            @pl.when(s0 + 2 < nsteps)
            def _():
                copy(s0 + 2, rows0, sem0).start()

            copy(s0 + 1, rows1, sem1).wait()
            accs = chunk(rows1, accs)

            @pl.when(s0 + 3 < nsteps)
            def _():
                copy(s0 + 3, rows1, sem1).start()

            return accs

        accs = (jnp.zeros((L,), jnp.float32), jnp.zeros((L,), jnp.float32))
        accs = lax.fori_loop(0, giters, gstep, accs)
        total = accs[0]
        for a in accs[1:]:
            total = total + a
        accv[...] = total
        pltpu.sync_copy(accv, out.at[wid])

    return nh_partial


def kernel(output, adj):
    nbatch, n, d = output.shape
    nh = adj.shape[1]
    assert nh == NH and d % 32 == 0
    adj_flat = adj.reshape(-1)
    npw0, npw1 = 1344, 1792
    assert NS * (npw0 + npw1) >= n
    vlast = n * nh - (NS * npw0 + (NS - 1) * npw1) * nh
    assert 0 < vlast <= npw1 * nh and vlast % 16 == 0
    def _bf16_bits(x):
        u = jax.lax.bitcast_convert_type(x, jnp.uint32)
        return (u + jnp.uint32(0x7FFF) + ((u >> 16) & jnp.uint32(1))) >> 16

    table = jax.lax.bitcast_convert_type(
        _bf16_bits(output[0]) | (_bf16_bits(output[1]) << 16), jnp.int32)
    parts = _make_partial_kernel(nbatch, npw0, npw1, d, vlast)(table, adj_flat)
    denom = nbatch * n * (nh - 1) * d
    return jnp.sqrt(jnp.sum(parts) / jnp.float32(denom))


# speedup vs baseline: 1.1271x; 1.1271x over previous
import functools

import jax
import jax.numpy as jnp
from jax import lax
from jax.experimental import pallas as pl
from jax.experimental.pallas import tpu as pltpu
from jax.experimental.pallas import tpu_sc as plsc

NC = 2
NS = 16
NW = NC * NS
L = 16
CHUNK = 16
NH = 7
RPC = CHUNK * NH


def _unpack_bf16_pair(w):
    lo = lax.bitcast_convert_type(
        lax.shift_left(w, jnp.int32(16)), jnp.float32)
    hi = lax.bitcast_convert_type(w, jnp.float32)
    return lo, hi


@functools.lru_cache(maxsize=None)
def _make_partial_kernel(nbatch: int, npw0: int, npw1: int, d: int,
                         vlast: int):
    dw = d
    st0 = npw0 // CHUNK
    st1 = npw1 // CHUNK
    aw0 = npw0 * NH
    aw1 = npw1 * NH
    awords = max(aw0, aw1)
    assert st0 % 2 == 0 and st1 % 2 == 0
    assert aw0 % 8 == 0 and aw1 % 8 == 0

    mesh = plsc.VectorSubcoreMesh(core_axis_name="c", subcore_axis_name="s")

    @functools.partial(
        pl.kernel,
        mesh=mesh,
        out_type=jax.ShapeDtypeStruct((NW, L), jnp.float32),
        scratch_types=[
            pltpu.VMEM((awords,), jnp.int32),
            pltpu.VMEM((RPC, dw), jnp.int32),
            pltpu.VMEM((RPC, dw), jnp.int32),
            pltpu.VMEM((L,), jnp.float32),
            pltpu.SemaphoreType.DMA,
            pltpu.SemaphoreType.DMA,
        ],
    )
    def nh_partial(table, adjw, out, adjv, rows0, rows1, accv, sem0, sem1):
        c = lax.axis_index("c")
        s = lax.axis_index("s")
        wid = c * NS + s
        nsteps = jnp.where(c == 0, st0, st1)
        giters = jnp.where(c == 0, st0 // 2, st1 // 2)
        off = pl.multiple_of(
            jnp.where(c == 0, s * aw0, NS * aw0 + s * aw1), 8)

        @pl.when(wid < NW - 1)
        def _():
            pltpu.sync_copy(adjw.at[pl.ds(off, awords)], adjv)

        @pl.when(wid == NW - 1)
        def _():
            pltpu.sync_copy(
                adjw.at[pl.ds(off, vlast)], adjv.at[pl.ds(0, vlast)])
            zero = jnp.zeros((L,), jnp.int32)
            for t in range(vlast, awords, L):
                adjv[pl.ds(t, L)] = zero

        def copy(s, buf, sem):
            off = pl.multiple_of(s * RPC, 8)
            return pltpu.make_async_copy(
                table.at[adjv.at[pl.ds(off, RPC)]], buf, sem)

        copy(0, rows0, sem0).start()
        copy(1, rows1, sem1).start()

        def chunk(buf, accs):
            def node(i, accs):
                base = i * NH

                def jstep(j, accs):
                    a0, a1 = accs
                    col = pl.multiple_of(j * L, L)
                    c_lo, c_hi = _unpack_bf16_pair(buf[base, pl.ds(col, L)])
                    ds_lo = []
                    ds_hi = []
                    for k in range(1, NH):
                        n_lo, n_hi = _unpack_bf16_pair(
                            buf[base + k, pl.ds(col, L)])
                        ds_lo.append(jnp.abs(c_lo - n_lo))
                        ds_hi.append(jnp.abs(c_hi - n_hi))
                    t_lo = ((ds_lo[0] + ds_lo[1]) + (ds_lo[2] + ds_lo[3])
                            + (ds_lo[4] + ds_lo[5]))
                    t_hi = ((ds_hi[0] + ds_hi[1]) + (ds_hi[2] + ds_hi[3])
                            + (ds_hi[4] + ds_hi[5]))
                    return (a0 + t_lo, a1 + t_hi)

                return lax.fori_loop(0, dw // L, jstep, accs)
            return lax.fori_loop(0, CHUNK, node, accs)

        def gstep(g, accs):
            s0 = 2 * g
            copy(s0, rows0, sem0).wait()
            accs = chunk(rows0, accs)

            @pl.when(s0 + 2 < nsteps)
            def _():
                copy(s0 + 2, rows0, sem0).start()

            copy(s0 + 1, rows1, sem1).wait()
            accs = chunk(rows1, accs)

            @pl.when(s0 + 3 < nsteps)
            def _():
                copy(s0 + 3, rows1, sem1).start()

            return accs

        accs = (jnp.zeros((L,), jnp.float32), jnp.zeros((L,), jnp.float32))
        accs = lax.fori_loop(0, giters, gstep, accs)
        total = accs[0]
        for a in accs[1:]:
            total = total + a
        accv[...] = total
        pltpu.sync_copy(accv, out.at[wid])

    return nh_partial


def kernel(output, adj):
    nbatch, n, d = output.shape
    nh = adj.shape[1]
    assert nh == NH and d % 32 == 0
    adj_flat = adj.reshape(-1)
    npw0, npw1 = 1792, 1344
    assert NS * (npw0 + npw1) >= n
    vlast = n * nh - (NS * npw0 + (NS - 1) * npw1) * nh
    assert 0 < vlast <= npw1 * nh and vlast % 16 == 0
    def _bf16_bits(x):
        u = jax.lax.bitcast_convert_type(x, jnp.uint32)
        return (u + jnp.uint32(0x7FFF) + ((u >> 16) & jnp.uint32(1))) >> 16

    table = jax.lax.bitcast_convert_type(
        _bf16_bits(output[0]) | (_bf16_bits(output[1]) << 16), jnp.int32)
    parts = _make_partial_kernel(nbatch, npw0, npw1, d, vlast)(table, adj_flat)
    denom = nbatch * n * (nh - 1) * d
    return jnp.sqrt(jnp.sum(parts) / jnp.float32(denom))
